# Initial kernel scaffold; baseline (speedup 1.0000x reference)
import jax
import jax.numpy as jnp
from jax.experimental import pallas as pl


def kernel(x, adj_param, W_cheb, b_cheb, W_conv, b_conv):
    raise NotImplementedError("write your pallas kernel here")


# speedup vs baseline: 10.3981x; 10.3981x over previous
import jax
import jax.numpy as jnp
from jax.experimental import pallas as pl

_B, _CIN, _NN, _T = 64, 32, 22, 512
_COUT = 32


def _prep_body(adj_ref, adjT_ref, wcheb_ref, wchebT_ref, wct_ref,
               bcheb_row_ref, bconv_row_ref, x0_ref, m2_ref, aux_ref):
    nn = _NN
    adj = 0.5 * (jax.nn.sigmoid(adj_ref[...]) + jax.nn.sigmoid(adjT_ref[...]))
    row = jax.lax.broadcasted_iota(jnp.int32, (nn, nn), 0)
    col = jax.lax.broadcasted_iota(jnp.int32, (nn, nn), 1)
    adj = jnp.where(row == col, 0.0, adj)
    deg_c = jnp.sum(adj, axis=1, keepdims=True)
    dis_c = jnp.where(deg_c > 0, jax.lax.rsqrt(deg_c), 0.0)
    deg_r = jnp.sum(adj, axis=0, keepdims=True)
    dis_r = jnp.where(deg_r > 0, jax.lax.rsqrt(deg_r), 0.0)
    S = -(dis_c * adj * dis_r)

    x0 = x0_ref[...]
    z1 = jnp.dot(S, x0, preferred_element_type=jnp.float32)
    z2 = jnp.dot(S, z1, preferred_element_type=jnp.float32)
    d2 = (jnp.dot(z1, wchebT_ref[1], preferred_element_type=jnp.float32)
          + 2.0 * jnp.dot(z2, wchebT_ref[2], preferred_element_type=jnp.float32))

    wct = wct_ref[...]
    corr_row = jnp.sum(jnp.sum(wct * d2[:, None, :], axis=2), axis=0, keepdims=True)
    const_row = (jnp.sum(jnp.sum(wct * bcheb_row_ref[...][None, :, :], axis=2),
                         axis=0, keepdims=True)
                 + bconv_row_ref[...])
    aux_ref[0:1, :] = const_row
    aux_ref[1:2, :] = corr_row

    w03 = wcheb_ref[0] - wcheb_ref[2]
    for n in range(nn):
        m2_ref[n] = jnp.dot(wct[n], w03, preferred_element_type=jnp.float32)


def _main_body(x_ref, m2_ref, const_ref, corr_ref, out_ref):
    b = pl.program_id(0)
    xb = x_ref[0]
    acc = jnp.dot(m2_ref[...], xb, preferred_element_type=jnp.float32)
    acc = acc + const_ref[...]
    onehot_t0 = (jax.lax.broadcasted_iota(jnp.int32, (1, xb.shape[1]), 1) == 0
                 ).astype(jnp.float32)
    factor = jnp.where(b == 0, 1.0, 0.0)
    out_ref[0] = acc + factor * (corr_ref[...] * onehot_t0)


def kernel(x, adj_param, W_cheb, b_cheb, W_conv, b_conv):
    batch, cin, nn, t = x.shape
    cout = W_cheb.shape[1]
    wct = jnp.transpose(W_conv[..., 0], (2, 0, 1))
    x0 = x[0, :, :, 0].T

    m2_noc, aux = pl.pallas_call(
        _prep_body,
        out_shape=[
            jax.ShapeDtypeStruct((nn, cout, cin), jnp.float32),
            jax.ShapeDtypeStruct((2, cout), jnp.float32),
        ],
    )(adj_param, adj_param.T, W_cheb, jnp.transpose(W_cheb, (0, 2, 1)),
      wct, b_cheb.reshape(1, cout), b_conv.reshape(1, cout), x0)

    m2 = jnp.transpose(m2_noc, (1, 2, 0)).reshape(cout, cin * nn)
    const_col = aux[0].reshape(cout, 1)
    corr_col = aux[1].reshape(cout, 1)

    xr = x.reshape(batch, cin * nn, t)
    out = pl.pallas_call(
        _main_body,
        grid=(batch,),
        in_specs=[
            pl.BlockSpec((1, cin * nn, t), lambda b: (b, 0, 0)),
            pl.BlockSpec((cout, cin * nn), lambda b: (0, 0)),
            pl.BlockSpec((cout, 1), lambda b: (0, 0)),
            pl.BlockSpec((cout, 1), lambda b: (0, 0)),
        ],
        out_specs=pl.BlockSpec((1, cout, t), lambda b: (b, 0, 0)),
        out_shape=jax.ShapeDtypeStruct((batch, cout, t), jnp.float32),
    )(xr, m2, const_col, corr_col)

    return out[:, :, None, :]


# speedup vs baseline: 13.0648x; 1.2565x over previous
import jax
import jax.numpy as jnp
from jax.experimental import pallas as pl

_B, _CIN, _NN, _T = 64, 32, 22, 512
_COUT = 32


def _prep_body(adj_ref, adjT_ref, wcheb_ref, wchebT_ref, wct_ref,
               bcheb_row_ref, bconv_row_ref, x0_ref, m2_ref, aux_ref):
    nn = _NN
    adj = 0.5 * (jax.nn.sigmoid(adj_ref[...]) + jax.nn.sigmoid(adjT_ref[...]))
    row = jax.lax.broadcasted_iota(jnp.int32, (nn, nn), 0)
    col = jax.lax.broadcasted_iota(jnp.int32, (nn, nn), 1)
    adj = jnp.where(row == col, 0.0, adj)
    deg_c = jnp.sum(adj, axis=1, keepdims=True)
    dis_c = jnp.where(deg_c > 0, jax.lax.rsqrt(deg_c), 0.0)
    deg_r = jnp.sum(adj, axis=0, keepdims=True)
    dis_r = jnp.where(deg_r > 0, jax.lax.rsqrt(deg_r), 0.0)
    S = -(dis_c * adj * dis_r)

    x0 = x0_ref[...]
    z1 = jnp.dot(S, x0, preferred_element_type=jnp.float32)
    z2 = jnp.dot(S, z1, preferred_element_type=jnp.float32)
    d2 = (jnp.dot(z1, wchebT_ref[1], preferred_element_type=jnp.float32)
          + 2.0 * jnp.dot(z2, wchebT_ref[2], preferred_element_type=jnp.float32))

    wct = wct_ref[...]
    corr_row = jnp.sum(jnp.sum(wct * d2[:, None, :], axis=2), axis=0, keepdims=True)
    const_row = (jnp.sum(jnp.sum(wct * bcheb_row_ref[...][None, :, :], axis=2),
                         axis=0, keepdims=True)
                 + bconv_row_ref[...])
    aux_ref[0:1, :] = const_row
    aux_ref[1:2, :] = corr_row

    w03 = wcheb_ref[0] - wcheb_ref[2]
    for n in range(nn):
        m2_ref[n] = jnp.dot(wct[n], w03, preferred_element_type=jnp.float32)


def _main_body(x_ref, m2_ref, const_ref, corr_ref, out_ref):
    b = pl.program_id(0)
    cin, t = x_ref.shape[1], x_ref.shape[3]
    x3 = x_ref[0]
    m2 = m2_ref[...]
    acc = const_ref[...] * jnp.ones((1, t), jnp.float32)
    for c in range(cin):
        acc = acc + jnp.dot(m2[c], x3[c],
                            preferred_element_type=jnp.float32)
    onehot_t0 = (jax.lax.broadcasted_iota(jnp.int32, (1, t), 1) == 0
                 ).astype(jnp.float32)
    factor = jnp.where(b == 0, 1.0, 0.0)
    out_ref[0, :, 0, :] = acc + factor * (corr_ref[...] * onehot_t0)


def kernel(x, adj_param, W_cheb, b_cheb, W_conv, b_conv):
    batch, cin, nn, t = x.shape
    cout = W_cheb.shape[1]
    wct = jnp.transpose(W_conv[..., 0], (2, 0, 1))
    x0 = x[0, :, :, 0].T

    m2_noc, aux = pl.pallas_call(
        _prep_body,
        out_shape=[
            jax.ShapeDtypeStruct((nn, cout, cin), jnp.float32),
            jax.ShapeDtypeStruct((2, cout), jnp.float32),
        ],
    )(adj_param, adj_param.T, W_cheb, jnp.transpose(W_cheb, (0, 2, 1)),
      wct, b_cheb.reshape(1, cout), b_conv.reshape(1, cout), x0)

    m2 = jnp.transpose(m2_noc, (2, 1, 0))
    const_col = aux[0].reshape(cout, 1)
    corr_col = aux[1].reshape(cout, 1)

    out = pl.pallas_call(
        _main_body,
        grid=(batch,),
        in_specs=[
            pl.BlockSpec((1, cin, nn, t), lambda b: (b, 0, 0, 0)),
            pl.BlockSpec((cin, cout, nn), lambda b: (0, 0, 0)),
            pl.BlockSpec((cout, 1), lambda b: (0, 0)),
            pl.BlockSpec((cout, 1), lambda b: (0, 0)),
        ],
        out_specs=pl.BlockSpec((1, cout, 1, t), lambda b: (b, 0, 0, 0)),
        out_shape=jax.ShapeDtypeStruct((batch, cout, 1, t), jnp.float32),
    )(x, m2, const_col, corr_col)

    return out


# speedup vs baseline: 15.4956x; 1.1861x over previous
import jax
import jax.numpy as jnp
from jax.experimental import pallas as pl

_B, _CIN, _NN, _T = 64, 32, 22, 512
_COUT = 32


def _prep_body(adj_ref, adjT_ref, wcheb_ref, wchebT_ref, wct_ref,
               bcheb_row_ref, bconv_row_ref, x0_ref, m2_ref, aux_ref):
    nn = _NN
    adj = 0.5 * (jax.nn.sigmoid(adj_ref[...]) + jax.nn.sigmoid(adjT_ref[...]))
    row = jax.lax.broadcasted_iota(jnp.int32, (nn, nn), 0)
    col = jax.lax.broadcasted_iota(jnp.int32, (nn, nn), 1)
    adj = jnp.where(row == col, 0.0, adj)
    deg_c = jnp.sum(adj, axis=1, keepdims=True)
    dis_c = jnp.where(deg_c > 0, jax.lax.rsqrt(deg_c), 0.0)
    deg_r = jnp.sum(adj, axis=0, keepdims=True)
    dis_r = jnp.where(deg_r > 0, jax.lax.rsqrt(deg_r), 0.0)
    S = -(dis_c * adj * dis_r)

    x0 = x0_ref[...]
    z1 = jnp.dot(S, x0, preferred_element_type=jnp.float32)
    z2 = jnp.dot(S, z1, preferred_element_type=jnp.float32)
    d2 = (jnp.dot(z1, wchebT_ref[1], preferred_element_type=jnp.float32)
          + 2.0 * jnp.dot(z2, wchebT_ref[2], preferred_element_type=jnp.float32))

    wct = wct_ref[...]
    corr_row = jnp.sum(jnp.sum(wct * d2[:, None, :], axis=2), axis=0, keepdims=True)
    const_row = (jnp.sum(jnp.sum(wct * bcheb_row_ref[...][None, :, :], axis=2),
                         axis=0, keepdims=True)
                 + bconv_row_ref[...])
    aux_ref[0:1, :] = const_row
    aux_ref[1:2, :] = corr_row

    w03 = wcheb_ref[0] - wcheb_ref[2]
    for n in range(nn):
        m2_ref[n] = jnp.dot(wct[n], w03, preferred_element_type=jnp.float32)


def _main_body(x_ref, m2_ref, const_ref, corr_ref, out_ref):
    b = pl.program_id(0)
    bblk, cin, t = x_ref.shape[0], x_ref.shape[1], x_ref.shape[3]
    m2 = m2_ref[...]
    const = const_ref[...] * jnp.ones((1, t), jnp.float32)
    for i in range(bblk):
        x3 = x_ref[i]
        acc = const
        for c in range(cin):
            acc = acc + jnp.dot(m2[c], x3[c],
                                preferred_element_type=jnp.float32)
        if i == 0:
            onehot_t0 = (jax.lax.broadcasted_iota(jnp.int32, (1, t), 1) == 0
                         ).astype(jnp.float32)
            factor = jnp.where(b == 0, 1.0, 0.0)
            acc = acc + factor * (corr_ref[...] * onehot_t0)
        out_ref[i, :, 0, :] = acc


def kernel(x, adj_param, W_cheb, b_cheb, W_conv, b_conv):
    batch, cin, nn, t = x.shape
    cout = W_cheb.shape[1]
    wct = jnp.transpose(W_conv[..., 0], (2, 0, 1))
    x0 = x[0, :, :, 0].T

    m2_noc, aux = pl.pallas_call(
        _prep_body,
        out_shape=[
            jax.ShapeDtypeStruct((nn, cout, cin), jnp.float32),
            jax.ShapeDtypeStruct((2, cout), jnp.float32),
        ],
    )(adj_param, adj_param.T, W_cheb, jnp.transpose(W_cheb, (0, 2, 1)),
      wct, b_cheb.reshape(1, cout), b_conv.reshape(1, cout), x0)

    m2 = jnp.transpose(m2_noc, (2, 1, 0))
    const_col = aux[0].reshape(cout, 1)
    corr_col = aux[1].reshape(cout, 1)

    bblk = 4
    out = pl.pallas_call(
        _main_body,
        grid=(batch // bblk,),
        in_specs=[
            pl.BlockSpec((bblk, cin, nn, t), lambda b: (b, 0, 0, 0)),
            pl.BlockSpec((cin, cout, nn), lambda b: (0, 0, 0)),
            pl.BlockSpec((cout, 1), lambda b: (0, 0)),
            pl.BlockSpec((cout, 1), lambda b: (0, 0)),
        ],
        out_specs=pl.BlockSpec((bblk, cout, 1, t), lambda b: (b, 0, 0, 0)),
        out_shape=jax.ShapeDtypeStruct((batch, cout, 1, t), jnp.float32),
    )(x, m2, const_col, corr_col)

    return out


# speedup vs baseline: 52.6259x; 3.3962x over previous
import jax
import jax.numpy as jnp
from jax.experimental import pallas as pl

_B, _CIN, _NN, _T = 64, 32, 22, 512
_COUT = 32


def _prep_body(adj_ref, adjT_ref, wcheb_ref, wchebT_ref, wct_ref,
               bcheb_row_ref, bconv_row_ref, x0_ref, m2_ref, aux_ref):
    nn = _NN
    adj = 0.5 * (jax.nn.sigmoid(adj_ref[...]) + jax.nn.sigmoid(adjT_ref[...]))
    row = jax.lax.broadcasted_iota(jnp.int32, (nn, nn), 0)
    col = jax.lax.broadcasted_iota(jnp.int32, (nn, nn), 1)
    adj = jnp.where(row == col, 0.0, adj)
    deg_c = jnp.sum(adj, axis=1, keepdims=True)
    dis_c = jnp.where(deg_c > 0, jax.lax.rsqrt(deg_c), 0.0)
    deg_r = jnp.sum(adj, axis=0, keepdims=True)
    dis_r = jnp.where(deg_r > 0, jax.lax.rsqrt(deg_r), 0.0)
    S = -(dis_c * adj * dis_r)

    x0 = x0_ref[...]
    z1 = jnp.dot(S, x0, preferred_element_type=jnp.float32)
    z2 = jnp.dot(S, z1, preferred_element_type=jnp.float32)
    d2 = (jnp.dot(z1, wchebT_ref[1], preferred_element_type=jnp.float32)
          + 2.0 * jnp.dot(z2, wchebT_ref[2], preferred_element_type=jnp.float32))

    wct = wct_ref[...]
    corr_row = jnp.sum(jnp.sum(wct * d2[:, None, :], axis=2), axis=0, keepdims=True)
    const_row = (jnp.sum(jnp.sum(wct * bcheb_row_ref[...][None, :, :], axis=2),
                         axis=0, keepdims=True)
                 + bconv_row_ref[...])
    aux_ref[0:1, :] = const_row
    aux_ref[1:2, :] = corr_row

    w03 = wcheb_ref[0] - wcheb_ref[2]
    for n in range(nn):
        m2_ref[n] = jnp.dot(wct[n], w03, preferred_element_type=jnp.float32)


def _main_body(x_ref, m2_ref, const_ref, corr_ref, out_ref):
    b = pl.program_id(0)
    bblk, nn, t = x_ref.shape[0], x_ref.shape[1], x_ref.shape[3]
    m2 = m2_ref[...]
    const = const_ref[...] * jnp.ones((1, t), jnp.float32)
    for i in range(bblk):
        x3 = x_ref[i]
        acc = const
        for n in range(nn):
            acc = acc + jnp.dot(m2[n], x3[n],
                                preferred_element_type=jnp.float32)
        if i == 0:
            onehot_t0 = (jax.lax.broadcasted_iota(jnp.int32, (1, t), 1) == 0
                         ).astype(jnp.float32)
            factor = jnp.where(b == 0, 1.0, 0.0)
            acc = acc + factor * (corr_ref[...] * onehot_t0)
        out_ref[i, :, 0, :] = acc


def kernel(x, adj_param, W_cheb, b_cheb, W_conv, b_conv):
    batch, cin, nn, t = x.shape
    cout = W_cheb.shape[1]
    wct = jnp.transpose(W_conv[..., 0], (2, 0, 1))
    xt = jnp.transpose(x, (0, 2, 1, 3))
    x0 = xt[0, :, :, 0]

    m2_noc, aux = pl.pallas_call(
        _prep_body,
        out_shape=[
            jax.ShapeDtypeStruct((nn, cout, cin), jnp.float32),
            jax.ShapeDtypeStruct((2, cout), jnp.float32),
        ],
    )(adj_param, adj_param.T, W_cheb, jnp.transpose(W_cheb, (0, 2, 1)),
      wct, b_cheb.reshape(1, cout), b_conv.reshape(1, cout), x0)

    const_col = aux[0].reshape(cout, 1)
    corr_col = aux[1].reshape(cout, 1)

    bblk = 4
    out = pl.pallas_call(
        _main_body,
        grid=(batch // bblk,),
        in_specs=[
            pl.BlockSpec((bblk, nn, cin, t), lambda b: (b, 0, 0, 0)),
            pl.BlockSpec((nn, cout, cin), lambda b: (0, 0, 0)),
            pl.BlockSpec((cout, 1), lambda b: (0, 0)),
            pl.BlockSpec((cout, 1), lambda b: (0, 0)),
        ],
        out_specs=pl.BlockSpec((bblk, cout, 1, t), lambda b: (b, 0, 0, 0)),
        out_shape=jax.ShapeDtypeStruct((batch, cout, 1, t), jnp.float32),
    )(xt, m2_noc, const_col, corr_col)

    return out


# speedup vs baseline: 53.2008x; 1.0109x over previous
import jax
import jax.numpy as jnp
from jax.experimental import pallas as pl
from jax.experimental.pallas import tpu as pltpu

_NN = 22


def _body(x_ref, adj_ref, adjT_ref, wcheb_ref, wct_ref, wc3_ref,
          bcheb_ref, bconv_ref, x0t_ref, out_ref, m2_s, cc_s, corr_s):
    b = pl.program_id(0)
    bblk, nn, t = x_ref.shape[0], x_ref.shape[1], x_ref.shape[3]

    @pl.when(b == 0)
    def _prep():
        adj = 0.5 * (jax.nn.sigmoid(adj_ref[...]) + jax.nn.sigmoid(adjT_ref[...]))
        row = jax.lax.broadcasted_iota(jnp.int32, (nn, nn), 0)
        col = jax.lax.broadcasted_iota(jnp.int32, (nn, nn), 1)
        adj = jnp.where(row == col, 0.0, adj)
        deg_c = jnp.sum(adj, axis=1, keepdims=True)
        dis_c = jnp.where(deg_c > 0, jax.lax.rsqrt(deg_c), 0.0)
        deg_r = jnp.sum(adj, axis=0, keepdims=True)
        dis_r = jnp.where(deg_r > 0, jax.lax.rsqrt(deg_r), 0.0)
        s22 = -(dis_c * adj * dis_r)

        x0t = x0t_ref[...]
        z1t = jnp.dot(x0t, s22, preferred_element_type=jnp.float32)
        z2t = jnp.dot(z1t, s22, preferred_element_type=jnp.float32)
        w1 = wcheb_ref[1]
        w2 = wcheb_ref[2]
        d2t = (jnp.dot(w1, z1t, preferred_element_type=jnp.float32)
               + 2.0 * jnp.dot(w2, z2t, preferred_element_type=jnp.float32))
        wc3 = wc3_ref[...]
        corr_s[...] = jnp.sum(jnp.sum(wc3 * d2t[None, :, :], axis=2),
                              axis=1, keepdims=True)

        wct = wct_ref[...]
        bcheb = bcheb_ref[...]
        cc = bconv_ref[...]
        w03 = wcheb_ref[0] - wcheb_ref[2]
        for n in range(nn):
            m2_s[n] = jnp.dot(wct[n], w03, preferred_element_type=jnp.float32)
            cc = cc + jnp.dot(wct[n], bcheb, preferred_element_type=jnp.float32)
        cc_s[...] = cc

    m2 = m2_s[...]
    const = cc_s[...] * jnp.ones((1, t), jnp.float32)
    for i in range(bblk):
        x3 = x_ref[i]
        acc = const
        for n in range(nn):
            acc = acc + jnp.dot(m2[n], x3[n],
                                preferred_element_type=jnp.float32)
        if i == 0:
            onehot_t0 = (jax.lax.broadcasted_iota(jnp.int32, (1, t), 1) == 0
                         ).astype(jnp.float32)
            factor = jnp.where(b == 0, 1.0, 0.0)
            acc = acc + factor * (corr_s[...] * onehot_t0)
        out_ref[i, :, 0, :] = acc


def kernel(x, adj_param, W_cheb, b_cheb, W_conv, b_conv):
    batch, cin, nn, t = x.shape
    cout = W_cheb.shape[1]
    xt = jnp.transpose(x, (0, 2, 1, 3))
    wc3 = W_conv[..., 0]
    wct = jnp.transpose(wc3, (2, 0, 1))
    x0t = x[0, :, :, 0]

    bblk = 4
    out = pl.pallas_call(
        _body,
        grid=(batch // bblk,),
        in_specs=[
            pl.BlockSpec((bblk, nn, cin, t), lambda b: (b, 0, 0, 0)),
            pl.BlockSpec((nn, nn), lambda b: (0, 0)),
            pl.BlockSpec((nn, nn), lambda b: (0, 0)),
            pl.BlockSpec((3, cout, cin), lambda b: (0, 0, 0)),
            pl.BlockSpec((nn, cout, cout), lambda b: (0, 0, 0)),
            pl.BlockSpec((cout, cout, nn), lambda b: (0, 0, 0)),
            pl.BlockSpec((cout, 1), lambda b: (0, 0)),
            pl.BlockSpec((cout, 1), lambda b: (0, 0)),
            pl.BlockSpec((cin, nn), lambda b: (0, 0)),
        ],
        out_specs=pl.BlockSpec((bblk, cout, 1, t), lambda b: (b, 0, 0, 0)),
        out_shape=jax.ShapeDtypeStruct((batch, cout, 1, t), jnp.float32),
        scratch_shapes=[
            pltpu.VMEM((nn, cout, cout), jnp.float32),
            pltpu.VMEM((cout, 1), jnp.float32),
            pltpu.VMEM((cout, 1), jnp.float32),
        ],
    )(xt, adj_param, adj_param.T, W_cheb, wct, wc3,
      b_cheb.reshape(cout, 1), b_conv.reshape(cout, 1), x0t)

    return out
